# Initial kernel scaffold; baseline (speedup 1.0000x reference)
import jax
import jax.numpy as jnp
from jax.experimental import pallas as pl


def kernel(node_features, edge_features, edge_index, params):
    raise NotImplementedError("write your pallas kernel here")


# speedup vs baseline: 4.0339x; 4.0339x over previous
import functools

import jax
import jax.numpy as jnp
from jax import lax
from jax.experimental import pallas as pl
from jax.experimental.pallas import tpu as pltpu
from jax.experimental.pallas import tpu_sc as plsc

N_NODES = 10000
N_EDGES = 320000
HID = 128

NC = 2
NS = 16
NW = NC * NS

CH = 128
N_CHUNKS = N_EDGES // CH
WCH = 80
PC = NW * WCH
AGG_PAD = 10240
ROWS_T = AGG_PAD // NS

_f32 = jnp.float32

_DOT = functools.partial(jnp.dot, preferred_element_type=jnp.float32)


def _ln(x, g, b):
    mu = jnp.mean(x, axis=-1, keepdims=True)
    xc = x - mu
    var = jnp.mean(xc * xc, axis=-1, keepdims=True)
    return xc * jax.lax.rsqrt(var + 1e-5) * g + b


def _w_spec():
    return pl.BlockSpec((HID, HID), lambda i: (0, 0))


def _v_spec():
    return pl.BlockSpec((1, HID), lambda i: (0, 0))


def _rows_spec(blk, width=HID):
    return pl.BlockSpec((blk, width), lambda i: (i, 0))


_TC_PARAMS = pltpu.CompilerParams(dimension_semantics=("parallel",))


def _enc_body(x, w0, b0, w1, b1, w2, b2, g, be, o):
    h = jnp.maximum(_DOT(x[...], w0[...]) + b0[...], 0.0)
    h = jnp.maximum(_DOT(h, w1[...]) + b1[...], 0.0)
    h = _DOT(h, w2[...]) + b2[...]
    o[...] = _ln(h, g[...], be[...])


def _encoder(x, p, blk):
    (w0, b0), (w1, b1), (w2, b2) = p["layers"]
    g, be = p["ln"]
    n, din = x.shape
    grid = n // blk
    return pl.pallas_call(
        _enc_body,
        grid=(grid,),
        in_specs=[
            _rows_spec(blk, din),
            pl.BlockSpec((din, HID), lambda i: (0, 0)), _v_spec(),
            _w_spec(), _v_spec(),
            _w_spec(), _v_spec(),
            _v_spec(), _v_spec(),
        ],
        out_specs=_rows_spec(blk),
        out_shape=jax.ShapeDtypeStruct((n, HID), _f32),
        compiler_params=_TC_PARAMS,
    )(x, w0, b0.reshape(1, -1), w1, b1.reshape(1, -1), w2, b2.reshape(1, -1),
      g.reshape(1, -1), be.reshape(1, -1))


def _pq_body(x, wa, b1, wb, p, q):
    xx = x[...]
    p[...] = _DOT(xx, wa[...]) + b1[...]
    q[...] = _DOT(xx, wb[...])


def _pq(nf, wa, b1, wb, blk=2000):
    grid = N_NODES // blk
    out = jax.ShapeDtypeStruct((N_NODES, HID), _f32)
    return pl.pallas_call(
        _pq_body,
        grid=(grid,),
        in_specs=[_rows_spec(blk), _w_spec(), _v_spec(), _w_spec()],
        out_specs=(_rows_spec(blk), _rows_spec(blk)),
        out_shape=(out, out),
        compiler_params=_TC_PARAMS,
    )(nf, wa, b1.reshape(1, -1), wb)


def _edge_body(a, b, e, w1c, w2, b2, w3, b3, g, be, o):
    ev = e[...]
    h = jnp.maximum(a[...] + b[...] + _DOT(ev, w1c[...]), 0.0)
    h = jnp.maximum(_DOT(h, w2[...]) + b2[...], 0.0)
    h = _DOT(h, w3[...]) + b3[...]
    o[...] = _ln(h, g[...], be[...]) + ev


def _edge_block(bufa, bufb, ef, w1c, w2, b2, w3, b3, g, be, blk=2560):
    grid = N_EDGES // blk
    return pl.pallas_call(
        _edge_body,
        grid=(grid,),
        in_specs=[
            _rows_spec(blk), _rows_spec(blk), _rows_spec(blk),
            _w_spec(), _w_spec(), _v_spec(), _w_spec(), _v_spec(),
            _v_spec(), _v_spec(),
        ],
        out_specs=_rows_spec(blk),
        out_shape=jax.ShapeDtypeStruct((N_EDGES, HID), _f32),
        compiler_params=_TC_PARAMS,
    )(bufa, bufb, ef, w1c, w2, b2.reshape(1, -1), w3, b3.reshape(1, -1),
      g.reshape(1, -1), be.reshape(1, -1))


def _node_body(agg, x, w1a, w1b, b1, w2, b2, w3, b3, g, be, o):
    aggv = agg[0] + agg[1]
    xv = x[...]
    h = jnp.maximum(_DOT(aggv, w1a[...]) + _DOT(xv, w1b[...]) + b1[...], 0.0)
    h = jnp.maximum(_DOT(h, w2[...]) + b2[...], 0.0)
    h = _DOT(h, w3[...]) + b3[...]
    o[...] = _ln(h, g[...], be[...]) + xv


def _node_block(agg2, nf, w1a, w1b, b1, w2, b2, w3, b3, g, be, blk=2000):
    grid = N_NODES // blk
    return pl.pallas_call(
        _node_body,
        grid=(grid,),
        in_specs=[
            pl.BlockSpec((NC, blk, HID), lambda i: (0, i, 0)),
            _rows_spec(blk),
            _w_spec(), _w_spec(), _v_spec(),
            _w_spec(), _v_spec(), _w_spec(), _v_spec(),
            _v_spec(), _v_spec(),
        ],
        out_specs=_rows_spec(blk),
        out_shape=jax.ShapeDtypeStruct((N_NODES, HID), _f32),
        compiler_params=_TC_PARAMS,
    )(agg2, nf, w1a, w1b, b1.reshape(1, -1), w2, b2.reshape(1, -1),
      w3, b3.reshape(1, -1), g.reshape(1, -1), be.reshape(1, -1))


def _dec_body(x, w0, b0, w1, b1, w2, b2, o):
    h = jnp.maximum(_DOT(x[...], w0[...]) + b0[...], 0.0)
    h = jnp.maximum(_DOT(h, w1[...]) + b1[...], 0.0)
    o[...] = _DOT(h, w2[...]) + b2[...]


def _decoder(nf, p, blk=2000):
    (w0, b0), (w1, b1), (w2, b2) = p["layers"]
    dout = w2.shape[1]
    grid = N_NODES // blk
    return pl.pallas_call(
        _dec_body,
        grid=(grid,),
        in_specs=[
            _rows_spec(blk),
            _w_spec(), _v_spec(), _w_spec(), _v_spec(),
            pl.BlockSpec((HID, dout), lambda i: (0, 0)),
            pl.BlockSpec((1, dout), lambda i: (0, 0)),
        ],
        out_specs=_rows_spec(blk, dout),
        out_shape=jax.ShapeDtypeStruct((N_NODES, dout), _f32),
        compiler_params=_TC_PARAMS,
    )(nf, w0, b0.reshape(1, -1), w1, b1.reshape(1, -1), w2, b2.reshape(1, -1))


_MESH = plsc.VectorSubcoreMesh(core_axis_name="c", subcore_axis_name="s")


def _sc_gather(p, q, src3d, dst3d):
    out_t = jax.ShapeDtypeStruct((N_EDGES, HID), _f32)

    @functools.partial(
        pl.kernel,
        out_type=(out_t, out_t),
        mesh=_MESH,
        scratch_types=[
            pltpu.VMEM((WCH, 1, CH), jnp.int32),
            pltpu.VMEM((WCH, 1, CH), jnp.int32),
            pltpu.VMEM((2, CH, HID), _f32),
            pltpu.VMEM((2, CH, HID), _f32),
            pltpu.SemaphoreType.DMA,
            pltpu.SemaphoreType.DMA,
        ],
    )
    def k(p_hbm, q_hbm, src_hbm, dst_hbm, oa_hbm, ob_hbm,
          idxs, idxd, bufa, bufb, gsem, wsem):
        cid = lax.axis_index("c")
        sid = lax.axis_index("s")
        wid = sid * NC + cid
        cbase = wid * WCH
        pltpu.sync_copy(src_hbm.at[pl.ds(cbase, WCH)], idxs)
        pltpu.sync_copy(dst_hbm.at[pl.ds(cbase, WCH)], idxd)
        n_pairs = jnp.clip(N_CHUNKS - cbase, 0, WCH) // 2

        def pair_body(jj, carry):
            gd = []
            for s in range(2):
                j = jj * 2 + s
                gd.append(pltpu.async_copy(p_hbm.at[idxs.at[j, 0]], bufa.at[s], gsem))
                gd.append(pltpu.async_copy(q_hbm.at[idxd.at[j, 0]], bufb.at[s], gsem))
            for d in gd:
                d.wait()
            wd = []
            for s in range(2):
                row = (cbase + jj * 2 + s) * CH
                wd.append(pltpu.async_copy(bufa.at[s], oa_hbm.at[pl.ds(row, CH)], wsem))
                wd.append(pltpu.async_copy(bufb.at[s], ob_hbm.at[pl.ds(row, CH)], wsem))
            for d in wd:
                d.wait()
            return carry

        lax.fori_loop(0, n_pairs, pair_body, 0)

    return k(p, q, src3d, dst3d)


def _sc_segsum(ef, dst3d, zrows):
    out_t = jax.ShapeDtypeStruct((NC, AGG_PAD, HID), _f32)

    @functools.partial(
        pl.kernel,
        out_type=out_t,
        mesh=_MESH,
        scratch_types=[
            pltpu.VMEM((WCH, 1, CH), jnp.int32),
            pltpu.VMEM((2, CH, HID), _f32),
            pltpu.VMEM_SHARED((AGG_PAD, HID), _f32),
            pltpu.SemaphoreType.DMA,
        ],
    )
    def k(ef_hbm, dst_hbm, z_hbm, out_hbm, idxd, rows, agg, sem):
        cid = lax.axis_index("c")
        sid = lax.axis_index("s")
        wid = sid * NC + cid
        cbase = wid * WCH
        pltpu.sync_copy(dst_hbm.at[pl.ds(cbase, WCH)], idxd)
        n_pairs = jnp.clip(N_CHUNKS - cbase, 0, WCH) // 2

        pltpu.sync_copy(z_hbm, rows.at[0])
        for kk in range(ROWS_T // CH):
            pltpu.sync_copy(rows.at[0],
                            agg.at[pl.ds(sid * ROWS_T + kk * CH, CH)])
        plsc.subcore_barrier()

        def pair_body(jj, carry):
            gd = []
            for s in range(2):
                row = (cbase + jj * 2 + s) * CH
                gd.append(pltpu.async_copy(ef_hbm.at[pl.ds(row, CH)], rows.at[s], sem))
            for s in range(2):
                gd[s].wait()
                pltpu.sync_copy(rows.at[s], agg.at[idxd.at[jj * 2 + s, 0]], add=True)
            return carry

        lax.fori_loop(0, n_pairs, pair_body, 0)
        plsc.subcore_barrier()

        for kk in range(ROWS_T // CH):
            r0 = sid * ROWS_T + kk * CH
            pltpu.sync_copy(agg.at[pl.ds(r0, CH)], rows.at[0])
            pltpu.sync_copy(rows.at[0], out_hbm.at[cid, pl.ds(r0, CH)])

    return k(ef, dst3d, zrows)


def kernel(node_features, edge_features, edge_index, params):
    pad = jnp.zeros((PC * CH - N_EDGES,), jnp.int32)
    src3d = jnp.concatenate([edge_index[0], pad]).reshape(PC, 1, CH)
    dst3d = jnp.concatenate([edge_index[1], pad]).reshape(PC, 1, CH)
    zrows = jnp.zeros((CH, HID), _f32)

    nf = _encoder(node_features, params["node_enc"], 2000)
    ef = _encoder(edge_features, params["edge_enc"], 2560)

    for i in range(15):
        ep = params["edge_blocks"][i]
        (w1, b1), (w2, b2), (w3, b3) = ep["layers"]
        g, be = ep["ln"]
        w1a, w1b, w1c = w1[:HID], w1[HID:2 * HID], w1[2 * HID:]

        p, q = _pq(nf, w1a, b1, w1b)
        bufa, bufb = _sc_gather(p, q, src3d, dst3d)
        ef = _edge_block(bufa, bufb, ef, w1c, w2, b2, w3, b3, g, be)

        np_ = params["node_blocks"][i]
        (nw1, nb1), (nw2, nb2), (nw3, nb3) = np_["layers"]
        ng, nbe = np_["ln"]
        nw1a, nw1b = nw1[:HID], nw1[HID:]

        agg2 = _sc_segsum(ef, dst3d, zrows)
        nf = _node_block(agg2, nf, nw1a, nw1b, nb1, nw2, nb2, nw3, nb3, ng, nbe)

    return _decoder(nf, params["node_dec"])


# speedup vs baseline: 4.3946x; 1.0894x over previous
import functools

import jax
import jax.numpy as jnp
from jax import lax
from jax.experimental import pallas as pl
from jax.experimental.pallas import tpu as pltpu
from jax.experimental.pallas import tpu_sc as plsc

N_NODES = 10000
N_EDGES = 320000
HID = 128

NC = 2
NS = 16
NW = NC * NS

CH = 128
N_CHUNKS = N_EDGES // CH
N_HALF = 2
HWCH = 40
HPC = NW * HWCH
PC = N_HALF * HPC
HE = (HPC * CH, N_EDGES - HPC * CH)
AGG_PAD = 10240
ROWS_T = AGG_PAD // NS

_f32 = jnp.float32

_DOT = functools.partial(jnp.dot, preferred_element_type=jnp.float32)


def _ln(x, g, b):
    mu = jnp.mean(x, axis=-1, keepdims=True)
    xc = x - mu
    var = jnp.mean(xc * xc, axis=-1, keepdims=True)
    return xc * jax.lax.rsqrt(var + 1e-5) * g + b


def _w_spec():
    return pl.BlockSpec((HID, HID), lambda i: (0, 0))


def _v_spec():
    return pl.BlockSpec((1, HID), lambda i: (0, 0))


def _rows_spec(blk, width=HID):
    return pl.BlockSpec((blk, width), lambda i: (i, 0))


_TC_PARAMS = pltpu.CompilerParams(dimension_semantics=("parallel",))


def _enc_body(x, w0, b0, w1, b1, w2, b2, g, be, o):
    h = jnp.maximum(_DOT(x[...], w0[...]) + b0[...], 0.0)
    h = jnp.maximum(_DOT(h, w1[...]) + b1[...], 0.0)
    h = _DOT(h, w2[...]) + b2[...]
    o[...] = _ln(h, g[...], be[...])


def _encoder(x, p, blk):
    (w0, b0), (w1, b1), (w2, b2) = p["layers"]
    g, be = p["ln"]
    n, din = x.shape
    grid = n // blk
    return pl.pallas_call(
        _enc_body,
        grid=(grid,),
        in_specs=[
            _rows_spec(blk, din),
            pl.BlockSpec((din, HID), lambda i: (0, 0)), _v_spec(),
            _w_spec(), _v_spec(),
            _w_spec(), _v_spec(),
            _v_spec(), _v_spec(),
        ],
        out_specs=_rows_spec(blk),
        out_shape=jax.ShapeDtypeStruct((n, HID), _f32),
        compiler_params=_TC_PARAMS,
    )(x, w0, b0.reshape(1, -1), w1, b1.reshape(1, -1), w2, b2.reshape(1, -1),
      g.reshape(1, -1), be.reshape(1, -1))


def _pq_body(x, wa, b1, wb, p, q):
    xx = x[...]
    p[...] = _DOT(xx, wa[...]) + b1[...]
    q[...] = _DOT(xx, wb[...])


def _pq(nf, wa, b1, wb, blk=2000):
    grid = N_NODES // blk
    out = jax.ShapeDtypeStruct((N_NODES, HID), _f32)
    return pl.pallas_call(
        _pq_body,
        grid=(grid,),
        in_specs=[_rows_spec(blk), _w_spec(), _v_spec(), _w_spec()],
        out_specs=(_rows_spec(blk), _rows_spec(blk)),
        out_shape=(out, out),
        compiler_params=_TC_PARAMS,
    )(nf, wa, b1.reshape(1, -1), wb)


def _edge_body(a, b, e, w1c, w2, b2, w3, b3, g, be, o):
    ev = e[...]
    h = jnp.maximum(a[...] + b[...] + _DOT(ev, w1c[...]), 0.0)
    h = jnp.maximum(_DOT(h, w2[...]) + b2[...], 0.0)
    h = _DOT(h, w3[...]) + b3[...]
    o[...] = _ln(h, g[...], be[...]) + ev


def _edge_block(bufa, bufb, ef, w1c, w2, b2, w3, b3, g, be, blk=2560):
    n = ef.shape[0]
    grid = n // blk
    return pl.pallas_call(
        _edge_body,
        grid=(grid,),
        in_specs=[
            _rows_spec(blk), _rows_spec(blk), _rows_spec(blk),
            _w_spec(), _w_spec(), _v_spec(), _w_spec(), _v_spec(),
            _v_spec(), _v_spec(),
        ],
        out_specs=_rows_spec(blk),
        out_shape=jax.ShapeDtypeStruct((n, HID), _f32),
        compiler_params=_TC_PARAMS,
    )(bufa, bufb, ef, w1c, w2, b2.reshape(1, -1), w3, b3.reshape(1, -1),
      g.reshape(1, -1), be.reshape(1, -1))


def _node_body(agg0, agg1, x, w1a, w1b, b1, w2, b2, w3, b3, g, be, o):
    aggv = (agg0[0] + agg0[1]) + (agg1[0] + agg1[1])
    xv = x[...]
    h = jnp.maximum(_DOT(aggv, w1a[...]) + _DOT(xv, w1b[...]) + b1[...], 0.0)
    h = jnp.maximum(_DOT(h, w2[...]) + b2[...], 0.0)
    h = _DOT(h, w3[...]) + b3[...]
    o[...] = _ln(h, g[...], be[...]) + xv


def _node_block(agg_h0, agg_h1, nf, w1a, w1b, b1, w2, b2, w3, b3, g, be,
                blk=2000):
    grid = N_NODES // blk
    agg_spec = pl.BlockSpec((NC, blk, HID), lambda i: (0, i, 0))
    return pl.pallas_call(
        _node_body,
        grid=(grid,),
        in_specs=[
            agg_spec, agg_spec,
            _rows_spec(blk),
            _w_spec(), _w_spec(), _v_spec(),
            _w_spec(), _v_spec(), _w_spec(), _v_spec(),
            _v_spec(), _v_spec(),
        ],
        out_specs=_rows_spec(blk),
        out_shape=jax.ShapeDtypeStruct((N_NODES, HID), _f32),
        compiler_params=_TC_PARAMS,
    )(agg_h0, agg_h1, nf, w1a, w1b, b1.reshape(1, -1), w2, b2.reshape(1, -1),
      w3, b3.reshape(1, -1), g.reshape(1, -1), be.reshape(1, -1))


def _dec_body(x, w0, b0, w1, b1, w2, b2, o):
    h = jnp.maximum(_DOT(x[...], w0[...]) + b0[...], 0.0)
    h = jnp.maximum(_DOT(h, w1[...]) + b1[...], 0.0)
    o[...] = _DOT(h, w2[...]) + b2[...]


def _decoder(nf, p, blk=2000):
    (w0, b0), (w1, b1), (w2, b2) = p["layers"]
    dout = w2.shape[1]
    grid = N_NODES // blk
    return pl.pallas_call(
        _dec_body,
        grid=(grid,),
        in_specs=[
            _rows_spec(blk),
            _w_spec(), _v_spec(), _w_spec(), _v_spec(),
            pl.BlockSpec((HID, dout), lambda i: (0, 0)),
            pl.BlockSpec((1, dout), lambda i: (0, 0)),
        ],
        out_specs=_rows_spec(blk, dout),
        out_shape=jax.ShapeDtypeStruct((N_NODES, dout), _f32),
        compiler_params=_TC_PARAMS,
    )(nf, w0, b0.reshape(1, -1), w1, b1.reshape(1, -1), w2, b2.reshape(1, -1))


@functools.cache
def _mesh():
    return plsc.VectorSubcoreMesh(core_axis_name="c", subcore_axis_name="s",
                                  num_cores=NC, num_subcores=NS)


def _sc_gather(p, q, src3d_h, dst3d_h, hbase, he):
    out_t = jax.ShapeDtypeStruct((he, HID), _f32)

    @functools.partial(
        pl.kernel,
        out_type=(out_t, out_t),
        mesh=_mesh(),
        scratch_types=[
            pltpu.VMEM((HWCH, 1, CH), jnp.int32),
            pltpu.VMEM((HWCH, 1, CH), jnp.int32),
            pltpu.VMEM((2, CH, HID), _f32),
            pltpu.VMEM((2, CH, HID), _f32),
            pltpu.SemaphoreType.DMA,
            pltpu.SemaphoreType.DMA,
        ],
    )
    def k(p_hbm, q_hbm, src_hbm, dst_hbm, oa_hbm, ob_hbm,
          idxs, idxd, bufa, bufb, gsem, wsem):
        cid = lax.axis_index("c")
        sid = lax.axis_index("s")
        wid = sid * NC + cid
        cbase = wid * HWCH
        pltpu.sync_copy(src_hbm.at[pl.ds(cbase, HWCH)], idxs)
        pltpu.sync_copy(dst_hbm.at[pl.ds(cbase, HWCH)], idxd)
        n_pairs = jnp.clip(N_CHUNKS - (hbase + cbase), 0, HWCH) // 2

        def pair_body(jj, carry):
            gd = []
            for s in range(2):
                j = jj * 2 + s
                gd.append(pltpu.async_copy(p_hbm.at[idxs.at[j, 0]], bufa.at[s], gsem))
                gd.append(pltpu.async_copy(q_hbm.at[idxd.at[j, 0]], bufb.at[s], gsem))
            for d in gd:
                d.wait()
            wd = []
            for s in range(2):
                row = (cbase + jj * 2 + s) * CH
                wd.append(pltpu.async_copy(bufa.at[s], oa_hbm.at[pl.ds(row, CH)], wsem))
                wd.append(pltpu.async_copy(bufb.at[s], ob_hbm.at[pl.ds(row, CH)], wsem))
            for d in wd:
                d.wait()
            return carry

        lax.fori_loop(0, n_pairs, pair_body, 0)

    return k(p, q, src3d_h, dst3d_h)


def _sc_segsum(ef_h, dst3d_h, zrows, hbase):
    out_t = jax.ShapeDtypeStruct((NC, AGG_PAD, HID), _f32)

    @functools.partial(
        pl.kernel,
        out_type=out_t,
        mesh=_mesh(),
        scratch_types=[
            pltpu.VMEM((HWCH, 1, CH), jnp.int32),
            pltpu.VMEM((2, CH, HID), _f32),
            pltpu.VMEM_SHARED((AGG_PAD, HID), _f32),
            pltpu.SemaphoreType.DMA,
        ],
    )
    def k(ef_hbm, dst_hbm, z_hbm, out_hbm, idxd, rows, agg, sem):
        cid = lax.axis_index("c")
        sid = lax.axis_index("s")
        wid = sid * NC + cid
        cbase = wid * HWCH
        pltpu.sync_copy(dst_hbm.at[pl.ds(cbase, HWCH)], idxd)
        n_pairs = jnp.clip(N_CHUNKS - (hbase + cbase), 0, HWCH) // 2

        pltpu.sync_copy(z_hbm, rows.at[0])
        for kk in range(ROWS_T // CH):
            pltpu.sync_copy(rows.at[0],
                            agg.at[pl.ds(sid * ROWS_T + kk * CH, CH)])
        plsc.subcore_barrier()

        def pair_body(jj, carry):
            gd = []
            for s in range(2):
                row = (cbase + jj * 2 + s) * CH
                gd.append(pltpu.async_copy(ef_hbm.at[pl.ds(row, CH)], rows.at[s], sem))
            for s in range(2):
                gd[s].wait()
                pltpu.sync_copy(rows.at[s], agg.at[idxd.at[jj * 2 + s, 0]], add=True)
            return carry

        lax.fori_loop(0, n_pairs, pair_body, 0)
        plsc.subcore_barrier()

        for kk in range(ROWS_T // CH):
            r0 = sid * ROWS_T + kk * CH
            pltpu.sync_copy(agg.at[pl.ds(r0, CH)], rows.at[0])
            pltpu.sync_copy(rows.at[0], out_hbm.at[cid, pl.ds(r0, CH)])

    return k(ef_h, dst3d_h, zrows)


def kernel(node_features, edge_features, edge_index, params):
    pad = jnp.zeros((PC * CH - N_EDGES,), jnp.int32)
    src3d = jnp.concatenate([edge_index[0], pad]).reshape(PC, 1, CH)
    dst3d = jnp.concatenate([edge_index[1], pad]).reshape(PC, 1, CH)
    src_h = (src3d[:HPC], src3d[HPC:])
    dst_h = (dst3d[:HPC], dst3d[HPC:])
    zrows = jnp.zeros((CH, HID), _f32)

    nf = _encoder(node_features, params["node_enc"], 2000)
    ef = [_encoder(edge_features[:HE[0]], params["edge_enc"], 2560),
          _encoder(edge_features[HE[0]:], params["edge_enc"], 2560)]

    for i in range(15):
        ep = params["edge_blocks"][i]
        (w1, b1), (w2, b2), (w3, b3) = ep["layers"]
        g, be = ep["ln"]
        w1a, w1b, w1c = w1[:HID], w1[HID:2 * HID], w1[2 * HID:]

        p, q = _pq(nf, w1a, b1, w1b)
        bufs = [_sc_gather(p, q, src_h[h], dst_h[h], h * HPC, HE[h])
                for h in range(N_HALF)]
        ef = [_edge_block(bufs[h][0], bufs[h][1], ef[h], w1c, w2, b2,
                          w3, b3, g, be)
              for h in range(N_HALF)]

        np_ = params["node_blocks"][i]
        (nw1, nb1), (nw2, nb2), (nw3, nb3) = np_["layers"]
        ng, nbe = np_["ln"]
        nw1a, nw1b = nw1[:HID], nw1[HID:]

        aggs = [_sc_segsum(ef[h], dst_h[h], zrows, h * HPC)
                for h in range(N_HALF)]
        nf = _node_block(aggs[0], aggs[1], nf, nw1a, nw1b, nb1,
                         nw2, nb2, nw3, nb3, ng, nbe)

    return _decoder(nf, params["node_dec"])


# speedup vs baseline: 4.5390x; 1.0329x over previous
import functools

import jax
import jax.numpy as jnp
from jax import lax
from jax.experimental import pallas as pl
from jax.experimental.pallas import tpu as pltpu
from jax.experimental.pallas import tpu_sc as plsc

N_NODES = 10000
N_EDGES = 320000
HID = 128

NC = 2
NS = 16
NW = NC * NS

CH = 128
N_CHUNKS = N_EDGES // CH
N_HALF = 2
HWCH = 40
HPC = NW * HWCH
PC = N_HALF * HPC
HE = (HPC * CH, N_EDGES - HPC * CH)
AGG_PAD = 10240
ROWS_T = AGG_PAD // NS

_f32 = jnp.float32

_DOT = functools.partial(jnp.dot, preferred_element_type=jnp.float32)


def _ln(x, g, b):
    mu = jnp.mean(x, axis=-1, keepdims=True)
    xc = x - mu
    var = jnp.mean(xc * xc, axis=-1, keepdims=True)
    return xc * jax.lax.rsqrt(var + 1e-5) * g + b


def _w_spec():
    return pl.BlockSpec((HID, HID), lambda i: (0, 0))


def _v_spec():
    return pl.BlockSpec((1, HID), lambda i: (0, 0))


def _rows_spec(blk, width=HID):
    return pl.BlockSpec((blk, width), lambda i: (i, 0))


_TC_PARAMS = pltpu.CompilerParams(dimension_semantics=("parallel",))


def _enc_body(x, w0, b0, w1, b1, w2, b2, g, be, o):
    h = jnp.maximum(_DOT(x[...], w0[...]) + b0[...], 0.0)
    h = jnp.maximum(_DOT(h, w1[...]) + b1[...], 0.0)
    h = _DOT(h, w2[...]) + b2[...]
    o[...] = _ln(h, g[...], be[...])


def _encoder(x, p, blk):
    (w0, b0), (w1, b1), (w2, b2) = p["layers"]
    g, be = p["ln"]
    n, din = x.shape
    grid = n // blk
    return pl.pallas_call(
        _enc_body,
        grid=(grid,),
        in_specs=[
            _rows_spec(blk, din),
            pl.BlockSpec((din, HID), lambda i: (0, 0)), _v_spec(),
            _w_spec(), _v_spec(),
            _w_spec(), _v_spec(),
            _v_spec(), _v_spec(),
        ],
        out_specs=_rows_spec(blk),
        out_shape=jax.ShapeDtypeStruct((n, HID), _f32),
        compiler_params=_TC_PARAMS,
    )(x, w0, b0.reshape(1, -1), w1, b1.reshape(1, -1), w2, b2.reshape(1, -1),
      g.reshape(1, -1), be.reshape(1, -1))


def _pq_body(x, wa, b1, wb, p, q):
    xx = x[...]
    p[...] = _DOT(xx, wa[...]) + b1[...]
    q[...] = _DOT(xx, wb[...])


def _pq(nf, wa, b1, wb, blk=2000):
    grid = N_NODES // blk
    out = jax.ShapeDtypeStruct((N_NODES, HID), _f32)
    return pl.pallas_call(
        _pq_body,
        grid=(grid,),
        in_specs=[_rows_spec(blk), _w_spec(), _v_spec(), _w_spec()],
        out_specs=(_rows_spec(blk), _rows_spec(blk)),
        out_shape=(out, out),
        compiler_params=_TC_PARAMS,
    )(nf, wa, b1.reshape(1, -1), wb)


def _edge_body(a, b, e, w1c, w2, b2, w3, b3, g, be, o):
    ev = e[...]
    h = jnp.maximum(a[...] + b[...] + _DOT(ev, w1c[...]), 0.0)
    h = jnp.maximum(_DOT(h, w2[...]) + b2[...], 0.0)
    h = _DOT(h, w3[...]) + b3[...]
    o[...] = _ln(h, g[...], be[...]) + ev


def _edge_block(bufa, bufb, ef, w1c, w2, b2, w3, b3, g, be, blk=2560):
    n = ef.shape[0]
    grid = n // blk
    return pl.pallas_call(
        _edge_body,
        grid=(grid,),
        in_specs=[
            _rows_spec(blk), _rows_spec(blk), _rows_spec(blk),
            _w_spec(), _w_spec(), _v_spec(), _w_spec(), _v_spec(),
            _v_spec(), _v_spec(),
        ],
        out_specs=_rows_spec(blk),
        out_shape=jax.ShapeDtypeStruct((n, HID), _f32),
        compiler_params=_TC_PARAMS,
    )(bufa, bufb, ef, w1c, w2, b2.reshape(1, -1), w3, b3.reshape(1, -1),
      g.reshape(1, -1), be.reshape(1, -1))


def _node_body(agg0, agg1, x, w1a, w1b, b1, w2, b2, w3, b3, g, be, o):
    aggv = (agg0[0] + agg0[1]) + (agg1[0] + agg1[1])
    xv = x[...]
    h = jnp.maximum(_DOT(aggv, w1a[...]) + _DOT(xv, w1b[...]) + b1[...], 0.0)
    h = jnp.maximum(_DOT(h, w2[...]) + b2[...], 0.0)
    h = _DOT(h, w3[...]) + b3[...]
    o[...] = _ln(h, g[...], be[...]) + xv


def _node_block(agg_h0, agg_h1, nf, w1a, w1b, b1, w2, b2, w3, b3, g, be,
                blk=2000):
    grid = N_NODES // blk
    agg_spec = pl.BlockSpec((NC, blk, HID), lambda i: (0, i, 0))
    return pl.pallas_call(
        _node_body,
        grid=(grid,),
        in_specs=[
            agg_spec, agg_spec,
            _rows_spec(blk),
            _w_spec(), _w_spec(), _v_spec(),
            _w_spec(), _v_spec(), _w_spec(), _v_spec(),
            _v_spec(), _v_spec(),
        ],
        out_specs=_rows_spec(blk),
        out_shape=jax.ShapeDtypeStruct((N_NODES, HID), _f32),
        compiler_params=_TC_PARAMS,
    )(agg_h0, agg_h1, nf, w1a, w1b, b1.reshape(1, -1), w2, b2.reshape(1, -1),
      w3, b3.reshape(1, -1), g.reshape(1, -1), be.reshape(1, -1))


def _dec_body(x, w0, b0, w1, b1, w2, b2, o):
    h = jnp.maximum(_DOT(x[...], w0[...]) + b0[...], 0.0)
    h = jnp.maximum(_DOT(h, w1[...]) + b1[...], 0.0)
    o[...] = _DOT(h, w2[...]) + b2[...]


def _decoder(nf, p, blk=2000):
    (w0, b0), (w1, b1), (w2, b2) = p["layers"]
    dout = w2.shape[1]
    grid = N_NODES // blk
    return pl.pallas_call(
        _dec_body,
        grid=(grid,),
        in_specs=[
            _rows_spec(blk),
            _w_spec(), _v_spec(), _w_spec(), _v_spec(),
            pl.BlockSpec((HID, dout), lambda i: (0, 0)),
            pl.BlockSpec((1, dout), lambda i: (0, 0)),
        ],
        out_specs=_rows_spec(blk, dout),
        out_shape=jax.ShapeDtypeStruct((N_NODES, dout), _f32),
        compiler_params=_TC_PARAMS,
    )(nf, w0, b0.reshape(1, -1), w1, b1.reshape(1, -1), w2, b2.reshape(1, -1))


@functools.cache
def _mesh():
    return plsc.VectorSubcoreMesh(core_axis_name="c", subcore_axis_name="s",
                                  num_cores=NC, num_subcores=NS)


def _sc_gather(p, q, src3d_h, dst3d_h, hbase, he):
    out_t = jax.ShapeDtypeStruct((he, HID), _f32)

    @functools.partial(
        pl.kernel,
        out_type=(out_t, out_t),
        mesh=_mesh(),
        scratch_types=[
            pltpu.VMEM((HWCH, 1, CH), jnp.int32),
            pltpu.VMEM((HWCH, 1, CH), jnp.int32),
            pltpu.VMEM((3, CH, HID), _f32),
            pltpu.VMEM((3, CH, HID), _f32),
            pltpu.SemaphoreType.DMA,
            pltpu.SemaphoreType.DMA,
        ],
    )
    def k(p_hbm, q_hbm, src_hbm, dst_hbm, oa_hbm, ob_hbm,
          idxs, idxd, bufa, bufb, gsem, wsem):
        cid = lax.axis_index("c")
        sid = lax.axis_index("s")
        wid = sid * NC + cid
        cbase = wid * HWCH
        pltpu.sync_copy(src_hbm.at[pl.ds(cbase, HWCH)], idxs)
        pltpu.sync_copy(dst_hbm.at[pl.ds(cbase, HWCH)], idxd)
        n_real = jnp.clip(N_CHUNKS - (hbase + cbase), 0, HWCH)

        def fire_g(j):
            slot = lax.rem(j, 3)
            pltpu.async_copy(p_hbm.at[idxs.at[j, 0]], bufa.at[slot], gsem)
            pltpu.async_copy(q_hbm.at[idxd.at[j, 0]], bufb.at[slot], gsem)

        for j0 in range(3):
            @pl.when(j0 < n_real)
            def _(j0=j0):
                fire_g(j0)

        def body(j, carry):
            slot = lax.rem(j, 3)
            pltpu.make_async_copy(p_hbm.at[pl.ds(0, CH)], bufa.at[slot], gsem).wait()
            pltpu.make_async_copy(q_hbm.at[pl.ds(0, CH)], bufb.at[slot], gsem).wait()
            row = (cbase + j) * CH
            da = pltpu.async_copy(bufa.at[slot], oa_hbm.at[pl.ds(row, CH)], wsem)
            db = pltpu.async_copy(bufb.at[slot], ob_hbm.at[pl.ds(row, CH)], wsem)
            da.wait()
            db.wait()

            @pl.when(j + 3 < n_real)
            def _():
                fire_g(j + 3)

            return carry

        lax.fori_loop(0, n_real, body, 0)

    return k(p, q, src3d_h, dst3d_h)


def _sc_segsum(ef_h, dst3d_h, zrows, hbase):
    out_t = jax.ShapeDtypeStruct((NC, AGG_PAD, HID), _f32)

    @functools.partial(
        pl.kernel,
        out_type=out_t,
        mesh=_mesh(),
        scratch_types=[
            pltpu.VMEM((HWCH, 1, CH), jnp.int32),
            pltpu.VMEM((2, CH, HID), _f32),
            pltpu.VMEM_SHARED((AGG_PAD, HID), _f32),
            pltpu.SemaphoreType.DMA,
        ],
    )
    def k(ef_hbm, dst_hbm, z_hbm, out_hbm, idxd, rows, agg, sem):
        cid = lax.axis_index("c")
        sid = lax.axis_index("s")
        wid = sid * NC + cid
        cbase = wid * HWCH
        pltpu.sync_copy(dst_hbm.at[pl.ds(cbase, HWCH)], idxd)
        n_real = jnp.clip(N_CHUNKS - (hbase + cbase), 0, HWCH)

        pltpu.sync_copy(z_hbm, rows.at[0])
        for kk in range(ROWS_T // CH):
            pltpu.sync_copy(rows.at[0],
                            agg.at[pl.ds(sid * ROWS_T + kk * CH, CH)])
        plsc.subcore_barrier()

        def fire_r(j):
            slot = lax.rem(j, 2)
            pltpu.async_copy(ef_hbm.at[pl.ds((cbase + j) * CH, CH)],
                             rows.at[slot], sem)

        for j0 in range(2):
            @pl.when(j0 < n_real)
            def _(j0=j0):
                fire_r(j0)

        def body(j, carry):
            slot = lax.rem(j, 2)
            pltpu.make_async_copy(ef_hbm.at[pl.ds(0, CH)], rows.at[slot], sem).wait()
            pltpu.sync_copy(rows.at[slot], agg.at[idxd.at[j, 0]], add=True)

            @pl.when(j + 2 < n_real)
            def _():
                fire_r(j + 2)

            return carry

        lax.fori_loop(0, n_real, body, 0)
        plsc.subcore_barrier()

        for kk in range(ROWS_T // CH):
            r0 = sid * ROWS_T + kk * CH
            pltpu.sync_copy(agg.at[pl.ds(r0, CH)], rows.at[0])
            pltpu.sync_copy(rows.at[0], out_hbm.at[cid, pl.ds(r0, CH)])

    return k(ef_h, dst3d_h, zrows)


def kernel(node_features, edge_features, edge_index, params):
    pad = jnp.zeros((PC * CH - N_EDGES,), jnp.int32)
    src3d = jnp.concatenate([edge_index[0], pad]).reshape(PC, 1, CH)
    dst3d = jnp.concatenate([edge_index[1], pad]).reshape(PC, 1, CH)
    src_h = (src3d[:HPC], src3d[HPC:])
    dst_h = (dst3d[:HPC], dst3d[HPC:])
    zrows = jnp.zeros((CH, HID), _f32)

    nf = _encoder(node_features, params["node_enc"], 2000)
    ef = [_encoder(edge_features[:HE[0]], params["edge_enc"], 2560),
          _encoder(edge_features[HE[0]:], params["edge_enc"], 2560)]

    for i in range(15):
        ep = params["edge_blocks"][i]
        (w1, b1), (w2, b2), (w3, b3) = ep["layers"]
        g, be = ep["ln"]
        w1a, w1b, w1c = w1[:HID], w1[HID:2 * HID], w1[2 * HID:]

        p, q = _pq(nf, w1a, b1, w1b)
        bufs = [_sc_gather(p, q, src_h[h], dst_h[h], h * HPC, HE[h])
                for h in range(N_HALF)]
        ef = [_edge_block(bufs[h][0], bufs[h][1], ef[h], w1c, w2, b2,
                          w3, b3, g, be)
              for h in range(N_HALF)]

        np_ = params["node_blocks"][i]
        (nw1, nb1), (nw2, nb2), (nw3, nb3) = np_["layers"]
        ng, nbe = np_["ln"]
        nw1a, nw1b = nw1[:HID], nw1[HID:]

        aggs = [_sc_segsum(ef[h], dst_h[h], zrows, h * HPC)
                for h in range(N_HALF)]
        nf = _node_block(aggs[0], aggs[1], nf, nw1a, nw1b, nb1,
                         nw2, nb2, nw3, nb3, ng, nbe)

    return _decoder(nf, params["node_dec"])


# speedup vs baseline: 4.5870x; 1.0106x over previous
import functools

import jax
import jax.numpy as jnp
from jax import lax
from jax.experimental import pallas as pl
from jax.experimental.pallas import tpu as pltpu
from jax.experimental.pallas import tpu_sc as plsc

N_NODES = 10000
N_EDGES = 320000
HID = 128

NC = 2
NS = 16
NW = NC * NS

CH = 128
N_CHUNKS = N_EDGES // CH
N_HALF = 2
HWCH = 40
HPC = NW * HWCH
PC = N_HALF * HPC
HE = (HPC * CH, N_EDGES - HPC * CH)
AGG_PAD = 10240
ROWS_T = AGG_PAD // NS

_f32 = jnp.float32

_DOT = functools.partial(jnp.dot, preferred_element_type=jnp.float32)


def _ln(x, g, b):
    mu = jnp.mean(x, axis=-1, keepdims=True)
    xc = x - mu
    var = jnp.mean(xc * xc, axis=-1, keepdims=True)
    return xc * jax.lax.rsqrt(var + 1e-5) * g + b


def _w_spec():
    return pl.BlockSpec((HID, HID), lambda i: (0, 0))


def _v_spec():
    return pl.BlockSpec((1, HID), lambda i: (0, 0))


def _rows_spec(blk, width=HID):
    return pl.BlockSpec((blk, width), lambda i: (i, 0))


_TC_PARAMS = pltpu.CompilerParams(dimension_semantics=("parallel",))


def _enc_body(x, w0, b0, w1, b1, w2, b2, g, be, o):
    h = jnp.maximum(_DOT(x[...], w0[...]) + b0[...], 0.0)
    h = jnp.maximum(_DOT(h, w1[...]) + b1[...], 0.0)
    h = _DOT(h, w2[...]) + b2[...]
    o[...] = _ln(h, g[...], be[...])


def _encoder(x, p, blk):
    (w0, b0), (w1, b1), (w2, b2) = p["layers"]
    g, be = p["ln"]
    n, din = x.shape
    grid = n // blk
    return pl.pallas_call(
        _enc_body,
        grid=(grid,),
        in_specs=[
            _rows_spec(blk, din),
            pl.BlockSpec((din, HID), lambda i: (0, 0)), _v_spec(),
            _w_spec(), _v_spec(),
            _w_spec(), _v_spec(),
            _v_spec(), _v_spec(),
        ],
        out_specs=_rows_spec(blk),
        out_shape=jax.ShapeDtypeStruct((n, HID), _f32),
        compiler_params=_TC_PARAMS,
    )(x, w0, b0.reshape(1, -1), w1, b1.reshape(1, -1), w2, b2.reshape(1, -1),
      g.reshape(1, -1), be.reshape(1, -1))


def _enc_pq_body(x, w0, b0, w1, b1, w2, b2, g, be, wa, b1e, wb, o, p, q):
    h = jnp.maximum(_DOT(x[...], w0[...]) + b0[...], 0.0)
    h = jnp.maximum(_DOT(h, w1[...]) + b1[...], 0.0)
    h = _DOT(h, w2[...]) + b2[...]
    nfv = _ln(h, g[...], be[...])
    o[...] = nfv
    p[...] = _DOT(nfv, wa[...]) + b1e[...]
    q[...] = _DOT(nfv, wb[...])


def _encoder_pq(x, pp, wa, b1e, wb, blk=2000):
    (w0, b0), (w1, b1), (w2, b2) = pp["layers"]
    g, be = pp["ln"]
    n, din = x.shape
    grid = n // blk
    out = jax.ShapeDtypeStruct((n, HID), _f32)
    return pl.pallas_call(
        _enc_pq_body,
        grid=(grid,),
        in_specs=[
            _rows_spec(blk, din),
            pl.BlockSpec((din, HID), lambda i: (0, 0)), _v_spec(),
            _w_spec(), _v_spec(),
            _w_spec(), _v_spec(),
            _v_spec(), _v_spec(),
            _w_spec(), _v_spec(), _w_spec(),
        ],
        out_specs=(_rows_spec(blk), _rows_spec(blk), _rows_spec(blk)),
        out_shape=(out, out, out),
        compiler_params=_TC_PARAMS,
    )(x, w0, b0.reshape(1, -1), w1, b1.reshape(1, -1), w2, b2.reshape(1, -1),
      g.reshape(1, -1), be.reshape(1, -1), wa, b1e.reshape(1, -1), wb)


def _edge_body(a, b, e, w1c, w2, b2, w3, b3, g, be, o):
    ev = e[...]
    h = jnp.maximum(a[...] + b[...] + _DOT(ev, w1c[...]), 0.0)
    h = jnp.maximum(_DOT(h, w2[...]) + b2[...], 0.0)
    h = _DOT(h, w3[...]) + b3[...]
    o[...] = _ln(h, g[...], be[...]) + ev


def _edge_block(bufa, bufb, ef, w1c, w2, b2, w3, b3, g, be, blk=2560):
    n = ef.shape[0]
    grid = n // blk
    return pl.pallas_call(
        _edge_body,
        grid=(grid,),
        in_specs=[
            _rows_spec(blk), _rows_spec(blk), _rows_spec(blk),
            _w_spec(), _w_spec(), _v_spec(), _w_spec(), _v_spec(),
            _v_spec(), _v_spec(),
        ],
        out_specs=_rows_spec(blk),
        out_shape=jax.ShapeDtypeStruct((n, HID), _f32),
        compiler_params=_TC_PARAMS,
    )(bufa, bufb, ef, w1c, w2, b2.reshape(1, -1), w3, b3.reshape(1, -1),
      g.reshape(1, -1), be.reshape(1, -1))


def _node_body(agg0, agg1, x, w1a, w1b, b1, w2, b2, w3, b3, g, be, o):
    aggv = (agg0[0] + agg0[1]) + (agg1[0] + agg1[1])
    xv = x[...]
    h = jnp.maximum(_DOT(aggv, w1a[...]) + _DOT(xv, w1b[...]) + b1[...], 0.0)
    h = jnp.maximum(_DOT(h, w2[...]) + b2[...], 0.0)
    h = _DOT(h, w3[...]) + b3[...]
    o[...] = _ln(h, g[...], be[...]) + xv


def _node_pq_body(agg0, agg1, x, w1a, w1b, b1, w2, b2, w3, b3, g, be,
                  wa, b1e, wb, o, p, q):
    aggv = (agg0[0] + agg0[1]) + (agg1[0] + agg1[1])
    xv = x[...]
    h = jnp.maximum(_DOT(aggv, w1a[...]) + _DOT(xv, w1b[...]) + b1[...], 0.0)
    h = jnp.maximum(_DOT(h, w2[...]) + b2[...], 0.0)
    h = _DOT(h, w3[...]) + b3[...]
    nfv = _ln(h, g[...], be[...]) + xv
    o[...] = nfv
    p[...] = _DOT(nfv, wa[...]) + b1e[...]
    q[...] = _DOT(nfv, wb[...])


def _node_block(agg_h0, agg_h1, nf, w1a, w1b, b1, w2, b2, w3, b3, g, be,
                nxt=None, blk=2000):
    grid = N_NODES // blk
    agg_spec = pl.BlockSpec((NC, blk, HID), lambda i: (0, i, 0))
    in_specs = [
        agg_spec, agg_spec,
        _rows_spec(blk),
        _w_spec(), _w_spec(), _v_spec(),
        _w_spec(), _v_spec(), _w_spec(), _v_spec(),
        _v_spec(), _v_spec(),
    ]
    args = [agg_h0, agg_h1, nf, w1a, w1b, b1.reshape(1, -1),
            w2, b2.reshape(1, -1), w3, b3.reshape(1, -1),
            g.reshape(1, -1), be.reshape(1, -1)]
    out = jax.ShapeDtypeStruct((N_NODES, HID), _f32)
    if nxt is None:
        return pl.pallas_call(
            _node_body, grid=(grid,), in_specs=in_specs,
            out_specs=_rows_spec(blk), out_shape=out,
            compiler_params=_TC_PARAMS,
        )(*args)
    wa, b1e, wb = nxt
    return pl.pallas_call(
        _node_pq_body, grid=(grid,),
        in_specs=in_specs + [_w_spec(), _v_spec(), _w_spec()],
        out_specs=(_rows_spec(blk), _rows_spec(blk), _rows_spec(blk)),
        out_shape=(out, out, out),
        compiler_params=_TC_PARAMS,
    )(*args, wa, b1e.reshape(1, -1), wb)


def _dec_body(x, w0, b0, w1, b1, w2, b2, o):
    h = jnp.maximum(_DOT(x[...], w0[...]) + b0[...], 0.0)
    h = jnp.maximum(_DOT(h, w1[...]) + b1[...], 0.0)
    o[...] = _DOT(h, w2[...]) + b2[...]


def _decoder(nf, p, blk=2000):
    (w0, b0), (w1, b1), (w2, b2) = p["layers"]
    dout = w2.shape[1]
    grid = N_NODES // blk
    return pl.pallas_call(
        _dec_body,
        grid=(grid,),
        in_specs=[
            _rows_spec(blk),
            _w_spec(), _v_spec(), _w_spec(), _v_spec(),
            pl.BlockSpec((HID, dout), lambda i: (0, 0)),
            pl.BlockSpec((1, dout), lambda i: (0, 0)),
        ],
        out_specs=_rows_spec(blk, dout),
        out_shape=jax.ShapeDtypeStruct((N_NODES, dout), _f32),
        compiler_params=_TC_PARAMS,
    )(nf, w0, b0.reshape(1, -1), w1, b1.reshape(1, -1), w2, b2.reshape(1, -1))


@functools.cache
def _mesh():
    return plsc.VectorSubcoreMesh(core_axis_name="c", subcore_axis_name="s",
                                  num_cores=NC, num_subcores=NS)


def _sc_gather(p, q, src3d_h, dst3d_h, hbase, he):
    out_t = jax.ShapeDtypeStruct((he, HID), _f32)

    @functools.partial(
        pl.kernel,
        out_type=(out_t, out_t),
        mesh=_mesh(),
        scratch_types=[
            pltpu.VMEM((HWCH, 1, CH), jnp.int32),
            pltpu.VMEM((HWCH, 1, CH), jnp.int32),
            pltpu.VMEM((3, CH, HID), _f32),
            pltpu.VMEM((3, CH, HID), _f32),
            pltpu.SemaphoreType.DMA,
            pltpu.SemaphoreType.DMA,
        ],
    )
    def k(p_hbm, q_hbm, src_hbm, dst_hbm, oa_hbm, ob_hbm,
          idxs, idxd, bufa, bufb, gsem, wsem):
        cid = lax.axis_index("c")
        sid = lax.axis_index("s")
        wid = sid * NC + cid
        cbase = wid * HWCH
        pltpu.sync_copy(src_hbm.at[pl.ds(cbase, HWCH)], idxs)
        pltpu.sync_copy(dst_hbm.at[pl.ds(cbase, HWCH)], idxd)
        n_real = jnp.clip(N_CHUNKS - (hbase + cbase), 0, HWCH)

        def fire_g(j):
            slot = lax.rem(j, 3)
            pltpu.async_copy(p_hbm.at[idxs.at[j, 0]], bufa.at[slot], gsem)
            pltpu.async_copy(q_hbm.at[idxd.at[j, 0]], bufb.at[slot], gsem)

        for j0 in range(3):
            @pl.when(j0 < n_real)
            def _(j0=j0):
                fire_g(j0)

        def body(j, carry):
            slot = lax.rem(j, 3)
            pltpu.make_async_copy(p_hbm.at[pl.ds(0, CH)], bufa.at[slot], gsem).wait()
            pltpu.make_async_copy(q_hbm.at[pl.ds(0, CH)], bufb.at[slot], gsem).wait()
            row = (cbase + j) * CH
            da = pltpu.async_copy(bufa.at[slot], oa_hbm.at[pl.ds(row, CH)], wsem)
            db = pltpu.async_copy(bufb.at[slot], ob_hbm.at[pl.ds(row, CH)], wsem)
            da.wait()
            db.wait()

            @pl.when(j + 3 < n_real)
            def _():
                fire_g(j + 3)

            return carry

        lax.fori_loop(0, n_real, body, 0)

    return k(p, q, src3d_h, dst3d_h)


def _sc_segsum(ef_h, dst3d_h, zrows, hbase):
    out_t = jax.ShapeDtypeStruct((NC, AGG_PAD, HID), _f32)

    @functools.partial(
        pl.kernel,
        out_type=out_t,
        mesh=_mesh(),
        scratch_types=[
            pltpu.VMEM((HWCH, 1, CH), jnp.int32),
            pltpu.VMEM((2, CH, HID), _f32),
            pltpu.VMEM_SHARED((AGG_PAD, HID), _f32),
            pltpu.SemaphoreType.DMA,
        ],
    )
    def k(ef_hbm, dst_hbm, z_hbm, out_hbm, idxd, rows, agg, sem):
        cid = lax.axis_index("c")
        sid = lax.axis_index("s")
        wid = sid * NC + cid
        cbase = wid * HWCH
        pltpu.sync_copy(dst_hbm.at[pl.ds(cbase, HWCH)], idxd)
        n_real = jnp.clip(N_CHUNKS - (hbase + cbase), 0, HWCH)

        pltpu.sync_copy(z_hbm, rows.at[0])
        for kk in range(ROWS_T // CH):
            pltpu.sync_copy(rows.at[0],
                            agg.at[pl.ds(sid * ROWS_T + kk * CH, CH)])
        plsc.subcore_barrier()

        def fire_r(j):
            slot = lax.rem(j, 2)
            pltpu.async_copy(ef_hbm.at[pl.ds((cbase + j) * CH, CH)],
                             rows.at[slot], sem)

        for j0 in range(2):
            @pl.when(j0 < n_real)
            def _(j0=j0):
                fire_r(j0)

        def body(j, carry):
            slot = lax.rem(j, 2)
            pltpu.make_async_copy(ef_hbm.at[pl.ds(0, CH)], rows.at[slot], sem).wait()
            pltpu.sync_copy(rows.at[slot], agg.at[idxd.at[j, 0]], add=True)

            @pl.when(j + 2 < n_real)
            def _():
                fire_r(j + 2)

            return carry

        lax.fori_loop(0, n_real, body, 0)
        plsc.subcore_barrier()

        for kk in range(ROWS_T // CH):
            r0 = sid * ROWS_T + kk * CH
            pltpu.sync_copy(agg.at[pl.ds(r0, CH)], rows.at[0])
            pltpu.sync_copy(rows.at[0], out_hbm.at[cid, pl.ds(r0, CH)])

    return k(ef_h, dst3d_h, zrows)


def kernel(node_features, edge_features, edge_index, params):
    pad = jnp.zeros((PC * CH - N_EDGES,), jnp.int32)
    src3d = jnp.concatenate([edge_index[0], pad]).reshape(PC, 1, CH)
    dst3d = jnp.concatenate([edge_index[1], pad]).reshape(PC, 1, CH)
    src_h = (src3d[:HPC], src3d[HPC:])
    dst_h = (dst3d[:HPC], dst3d[HPC:])
    zrows = jnp.zeros((CH, HID), _f32)

    def _pq_weights(i):
        (w1, b1) = params["edge_blocks"][i]["layers"][0]
        return w1[:HID], b1, w1[HID:2 * HID]

    nf, p, q = _encoder_pq(node_features, params["node_enc"], *_pq_weights(0))
    ef = [_encoder(edge_features[:HE[0]], params["edge_enc"], 2560),
          _encoder(edge_features[HE[0]:], params["edge_enc"], 2560)]

    for i in range(15):
        ep = params["edge_blocks"][i]
        (w1, b1), (w2, b2), (w3, b3) = ep["layers"]
        g, be = ep["ln"]
        w1c = w1[2 * HID:]

        bufs = [_sc_gather(p, q, src_h[h], dst_h[h], h * HPC, HE[h])
                for h in range(N_HALF)]
        ef = [_edge_block(bufs[h][0], bufs[h][1], ef[h], w1c, w2, b2,
                          w3, b3, g, be)
              for h in range(N_HALF)]

        np_ = params["node_blocks"][i]
        (nw1, nb1), (nw2, nb2), (nw3, nb3) = np_["layers"]
        ng, nbe = np_["ln"]
        nw1a, nw1b = nw1[:HID], nw1[HID:]

        aggs = [_sc_segsum(ef[h], dst_h[h], zrows, h * HPC)
                for h in range(N_HALF)]
        nxt = _pq_weights(i + 1) if i + 1 < 15 else None
        res = _node_block(aggs[0], aggs[1], nf, nw1a, nw1b, nb1,
                          nw2, nb2, nw3, nb3, ng, nbe, nxt=nxt)
        if nxt is None:
            nf = res
        else:
            nf, p, q = res

    return _decoder(nf, params["node_dec"])
